# Initial kernel scaffold; baseline (speedup 1.0000x reference)
import jax
import jax.numpy as jnp
from jax.experimental import pallas as pl


def kernel(user_idx, edge_index, x, W1, W2):
    raise NotImplementedError("write your pallas kernel here")


# speedup vs baseline: 6.7424x; 6.7424x over previous
import functools

import jax
import jax.numpy as jnp
from jax import lax
from jax.experimental import pallas as pl
from jax.experimental.pallas import tpu as pltpu
from jax.experimental.pallas import tpu_sc as plsc

_N_USERS = 5000
_N_NODES = 10000
_D = 128
_E = 320000
_B = 1024

_NP = 10240
_NW = 32
_NS = 16
_CH = 128
_CPT = 80
_EPT = _CPT * _CH
_EPAD = _NW * _EPT
_RPS = _NP // _NS

_mesh = plsc.VectorSubcoreMesh(core_axis_name="c", subcore_axis_name="s")


@functools.partial(
    pl.kernel,
    mesh=_mesh,
    out_type=jax.ShapeDtypeStruct((2, _NP, _D), jnp.float32),
    scratch_types=[
        pltpu.VMEM((_CPT, _CH), jnp.int32),
        pltpu.VMEM((_CH, _D), jnp.float32),
        pltpu.VMEM((_CH, _D), jnp.float32),
        pltpu.VMEM_SHARED((_NP, _D), jnp.float32),
    ],
)
def _deg_kernel(dst_hbm, out_hbm, didx, zb, ob, acc):
    c = lax.axis_index("c")
    s = lax.axis_index("s")
    wid = c * _NS + s
    zeros16 = jnp.zeros((16,), jnp.float32)
    one0 = jnp.where(lax.iota(jnp.int32, 16) == 0, 1.0, 0.0)

    def fill(j, _):
        for k in range(_D // 16):
            zb[j, pl.ds(k * 16, 16)] = zeros16
            ob[j, pl.ds(k * 16, 16)] = one0 if k == 0 else zeros16
        return 0
    lax.fori_loop(0, _CH, fill, 0)

    for i in range(_RPS // _CH):
        pltpu.sync_copy(zb, acc.at[pl.ds(s * _RPS + i * _CH, _CH)])
    plsc.subcore_barrier()

    pltpu.sync_copy(dst_hbm.at[wid], didx)

    def chunk(j, _):
        pltpu.sync_copy(ob, acc.at[didx.at[j]], add=True)
        return 0
    lax.fori_loop(0, _CPT, chunk, 0)

    plsc.subcore_barrier()
    pltpu.sync_copy(acc.at[pl.ds(s * _RPS, _RPS)],
                    out_hbm.at[c, pl.ds(s * _RPS, _RPS)])


@functools.partial(
    pl.kernel,
    mesh=_mesh,
    out_type=jax.ShapeDtypeStruct((2, _NP, _D), jnp.float32),
    scratch_types=[
        pltpu.VMEM((_CPT, _CH), jnp.int32),
        pltpu.VMEM((_CPT, _CH), jnp.int32),
        pltpu.VMEM((_CH, _D), jnp.float32),
        pltpu.VMEM_SHARED((_NP, _D), jnp.float32),
        pltpu.SemaphoreType.DMA,
    ],
)
def _conv_kernel(tab_hbm, src_hbm, dst_hbm, out_hbm, sidx, didx, rows, acc, gsem):
    c = lax.axis_index("c")
    s = lax.axis_index("s")
    wid = c * _NS + s
    zeros16 = jnp.zeros((16,), jnp.float32)

    def zrow(j, _):
        for k in range(_D // 16):
            rows[j, pl.ds(k * 16, 16)] = zeros16
        return 0
    lax.fori_loop(0, _CH, zrow, 0)
    for i in range(_RPS // _CH):
        pltpu.sync_copy(rows, acc.at[pl.ds(s * _RPS + i * _CH, _CH)])
    plsc.subcore_barrier()

    pltpu.sync_copy(src_hbm.at[wid], sidx)
    pltpu.sync_copy(dst_hbm.at[wid], didx)

    def chunk(j, _):
        cp = pltpu.make_async_copy(tab_hbm.at[sidx.at[j]], rows, gsem)
        cp.start()
        cp.wait()
        pltpu.sync_copy(rows, acc.at[didx.at[j]], add=True)
        return 0
    lax.fori_loop(0, _CPT, chunk, 0)

    plsc.subcore_barrier()
    pltpu.sync_copy(acc.at[pl.ds(s * _RPS, _RPS)],
                    out_hbm.at[c, pl.ds(s * _RPS, _RPS)])


@functools.partial(
    pl.kernel,
    mesh=_mesh,
    out_type=jax.ShapeDtypeStruct((_B, _D), jnp.float32),
    scratch_types=[
        pltpu.VMEM((_B // _NW,), jnp.int32),
        pltpu.VMEM((_B // _NW, _D), jnp.float32),
        pltpu.SemaphoreType.DMA,
    ],
)
def _ugather_kernel(tab_hbm, uidx_hbm, out_hbm, iv, rows, sem):
    c = lax.axis_index("c")
    s = lax.axis_index("s")
    wid = c * _NS + s
    bpw = _B // _NW
    pltpu.sync_copy(uidx_hbm.at[wid], iv)
    cp = pltpu.make_async_copy(tab_hbm.at[iv], rows, sem)
    cp.start()
    cp.wait()
    pltpu.sync_copy(rows, out_hbm.at[pl.ds(wid * bpw, bpw)])


_R = 1280
_G = _NP // _R


def _dinv_of(deg_ref):
    deg = deg_ref[0, :, 0:1] + deg_ref[1, :, 0:1]
    return lax.rsqrt(jnp.maximum(deg, 1.0))


def _prescale_body(deg_ref, x_ref, o_ref):
    o_ref[...] = x_ref[...] * _dinv_of(deg_ref)


def _layer1_body(deg_ref, p_ref, w_ref, o_ref):
    dinv = _dinv_of(deg_ref)
    y = (p_ref[0] + p_ref[1]) * dinv
    h = jnp.dot(y, w_ref[...], preferred_element_type=jnp.float32,
                precision=lax.Precision.HIGHEST)
    o_ref[...] = jnp.maximum(h, 0.0) * dinv


def _layer2_body(deg_ref, p_ref, w_ref, o_ref):
    dinv = _dinv_of(deg_ref)
    y = (p_ref[0] + p_ref[1]) * dinv
    o_ref[...] = jnp.dot(y, w_ref[...], preferred_element_type=jnp.float32,
                         precision=lax.Precision.HIGHEST)


_deg_spec = pl.BlockSpec((2, _R, _D), lambda i: (0, i, 0))
_p_spec = pl.BlockSpec((2, _R, _D), lambda i: (0, i, 0))
_row_spec = pl.BlockSpec((_R, _D), lambda i: (i, 0))
_w_spec = pl.BlockSpec((_D, _D), lambda i: (0, 0))
_out_shape = jax.ShapeDtypeStruct((_NP, _D), jnp.float32)


def _prescale(degp, xp):
    return pl.pallas_call(
        _prescale_body, grid=(_G,),
        in_specs=[_deg_spec, _row_spec],
        out_specs=_row_spec, out_shape=_out_shape,
    )(degp, xp)


def _layer(body, degp, parts, w):
    return pl.pallas_call(
        body, grid=(_G,),
        in_specs=[_deg_spec, _p_spec, _w_spec],
        out_specs=_row_spec, out_shape=_out_shape,
    )(degp, parts, w)


def kernel(user_idx, edge_index, x, W1, W2):
    src = edge_index[0].astype(jnp.int32)
    dst = edge_index[1].astype(jnp.int32)
    pad = _EPAD - _E
    srcp = jnp.concatenate(
        [src, jnp.full((pad,), _N_NODES, jnp.int32)]).reshape(_NW, _CPT, _CH)
    dstp = jnp.concatenate(
        [dst, jnp.full((pad,), _N_NODES, jnp.int32)]).reshape(_NW, _CPT, _CH)
    xp = jnp.pad(x, ((0, _NP - _N_NODES), (0, 0)))

    degp = _deg_kernel(dstp)
    xs = _prescale(degp, xp)
    p1 = _conv_kernel(xs, srcp, dstp)
    h1s = _layer(_layer1_body, degp, p1, W1)
    p2 = _conv_kernel(h1s, srcp, dstp)
    out = _layer(_layer2_body, degp, p2, W2)

    ue = _ugather_kernel(out, user_idx.astype(jnp.int32).reshape(_NW, _B // _NW))
    return ue, out[_N_USERS:_N_NODES]


# speedup vs baseline: 8.9301x; 1.3245x over previous
import functools

import jax
import jax.numpy as jnp
from jax import lax
from jax.experimental import pallas as pl
from jax.experimental.pallas import tpu as pltpu
from jax.experimental.pallas import tpu_sc as plsc

_N_USERS = 5000
_N_NODES = 10000
_D = 128
_E = 320000
_B = 1024

_NP = 10240
_NW = 32
_NS = 16
_CH = 128
_CPT = 80
_EPT = _CPT * _CH
_EPAD = _NW * _EPT
_RPS = _NP // _NS

_mesh = plsc.VectorSubcoreMesh(core_axis_name="c", subcore_axis_name="s")


@functools.partial(
    pl.kernel,
    mesh=_mesh,
    out_type=jax.ShapeDtypeStruct((2, _NP, _D), jnp.float32),
    scratch_types=[
        pltpu.VMEM((_CPT, _CH), jnp.int32),
        pltpu.VMEM((_CH, _D), jnp.float32),
        pltpu.VMEM((_CH, _D), jnp.float32),
        pltpu.VMEM_SHARED((_NP, _D), jnp.float32),
    ],
)
def _deg_kernel(dst_hbm, out_hbm, didx, zb, ob, acc):
    c = lax.axis_index("c")
    s = lax.axis_index("s")
    wid = c * _NS + s
    zeros16 = jnp.zeros((16,), jnp.float32)
    one0 = jnp.where(lax.iota(jnp.int32, 16) == 0, 1.0, 0.0)

    def fill(j, _):
        for k in range(_D // 16):
            zb[j, pl.ds(k * 16, 16)] = zeros16
            ob[j, pl.ds(k * 16, 16)] = one0 if k == 0 else zeros16
        return 0
    lax.fori_loop(0, _CH, fill, 0)

    for i in range(_RPS // _CH):
        pltpu.sync_copy(zb, acc.at[pl.ds(s * _RPS + i * _CH, _CH)])
    plsc.subcore_barrier()

    pltpu.sync_copy(dst_hbm.at[wid], didx)

    def chunk(j, _):
        pltpu.sync_copy(ob, acc.at[didx.at[j]], add=True)
        return 0
    lax.fori_loop(0, _CPT, chunk, 0)

    plsc.subcore_barrier()
    pltpu.sync_copy(acc.at[pl.ds(s * _RPS, _RPS)],
                    out_hbm.at[c, pl.ds(s * _RPS, _RPS)])


_CPT0 = 112
_CPT1 = 48
_NCH = _NS * (_CPT0 + _CPT1)
_EPAD2 = (_NCH + (_CPT0 - _CPT1)) * _CH


@functools.partial(
    pl.kernel,
    mesh=_mesh,
    out_type=jax.ShapeDtypeStruct((2, _NP, _D), jnp.float32),
    scratch_types=[
        pltpu.VMEM((_CPT0, _CH), jnp.int32),
        pltpu.VMEM((_CPT0, _CH), jnp.int32),
        pltpu.VMEM((_CH, _D), jnp.float32),
        pltpu.VMEM_SHARED((_NP, _D), jnp.float32),
        pltpu.SemaphoreType.DMA,
    ],
)
def _conv_kernel(tab_hbm, src_hbm, dst_hbm, out_hbm, sidx, didx, rows, acc, gsem):
    c = lax.axis_index("c")
    s = lax.axis_index("s")
    base = jnp.where(c == 0, s * _CPT0, _NS * _CPT0 + s * _CPT1)
    my_cpt = jnp.where(c == 0, _CPT0, _CPT1)
    zeros16 = jnp.zeros((16,), jnp.float32)

    def zrow(j, _):
        for k in range(_D // 16):
            rows[j, pl.ds(k * 16, 16)] = zeros16
        return 0
    lax.fori_loop(0, _CH, zrow, 0)
    for i in range(_RPS // _CH):
        pltpu.sync_copy(rows, acc.at[pl.ds(s * _RPS + i * _CH, _CH)])
    plsc.subcore_barrier()

    pltpu.sync_copy(src_hbm.at[pl.ds(base, _CPT0)], sidx)
    pltpu.sync_copy(dst_hbm.at[pl.ds(base, _CPT0)], didx)

    def chunk(j, _):
        @pl.when(j < my_cpt)
        def _():
            cp = pltpu.make_async_copy(tab_hbm.at[sidx.at[j]], rows, gsem)
            cp.start()
            cp.wait()
            pltpu.sync_copy(rows, acc.at[didx.at[j]], add=True)
        return 0
    lax.fori_loop(0, _CPT0, chunk, 0)

    plsc.subcore_barrier()
    pltpu.sync_copy(acc.at[pl.ds(s * _RPS, _RPS)],
                    out_hbm.at[c, pl.ds(s * _RPS, _RPS)])


@functools.partial(
    pl.kernel,
    mesh=_mesh,
    out_type=jax.ShapeDtypeStruct((_B, _D), jnp.float32),
    scratch_types=[
        pltpu.VMEM((_B // _NW,), jnp.int32),
        pltpu.VMEM((_B // _NW, _D), jnp.float32),
        pltpu.SemaphoreType.DMA,
    ],
)
def _ugather_kernel(tab_hbm, uidx_hbm, out_hbm, iv, rows, sem):
    c = lax.axis_index("c")
    s = lax.axis_index("s")
    wid = c * _NS + s
    bpw = _B // _NW
    pltpu.sync_copy(uidx_hbm.at[wid], iv)
    cp = pltpu.make_async_copy(tab_hbm.at[iv], rows, sem)
    cp.start()
    cp.wait()
    pltpu.sync_copy(rows, out_hbm.at[pl.ds(wid * bpw, bpw)])


_R = 1280
_G = _NP // _R


def _dinv_of(deg_ref):
    deg = deg_ref[0, :, 0:1] + deg_ref[1, :, 0:1]
    return lax.rsqrt(jnp.maximum(deg, 1.0))


def _prescale_body(deg_ref, x_ref, o_ref):
    o_ref[...] = x_ref[...] * _dinv_of(deg_ref)


def _layer1_body(deg_ref, p_ref, w_ref, o_ref):
    dinv = _dinv_of(deg_ref)
    y = (p_ref[0] + p_ref[1]) * dinv
    h = jnp.dot(y, w_ref[...], preferred_element_type=jnp.float32,
                precision=lax.Precision.HIGHEST)
    o_ref[...] = jnp.maximum(h, 0.0) * dinv


def _layer2_body(deg_ref, p_ref, w_ref, o_ref):
    dinv = _dinv_of(deg_ref)
    y = (p_ref[0] + p_ref[1]) * dinv
    o_ref[...] = jnp.dot(y, w_ref[...], preferred_element_type=jnp.float32,
                         precision=lax.Precision.HIGHEST)


_deg_spec = pl.BlockSpec((2, _R, _D), lambda i: (0, i, 0))
_p_spec = pl.BlockSpec((2, _R, _D), lambda i: (0, i, 0))
_row_spec = pl.BlockSpec((_R, _D), lambda i: (i, 0))
_w_spec = pl.BlockSpec((_D, _D), lambda i: (0, 0))
_out_shape = jax.ShapeDtypeStruct((_NP, _D), jnp.float32)


def _prescale(degp, xp):
    return pl.pallas_call(
        _prescale_body, grid=(_G,),
        in_specs=[_deg_spec, _row_spec],
        out_specs=_row_spec, out_shape=_out_shape,
    )(degp, xp)


def _layer(body, degp, parts, w):
    return pl.pallas_call(
        body, grid=(_G,),
        in_specs=[_deg_spec, _p_spec, _w_spec],
        out_specs=_row_spec, out_shape=_out_shape,
    )(degp, parts, w)


def kernel(user_idx, edge_index, x, W1, W2):
    src = edge_index[0].astype(jnp.int32)
    dst = edge_index[1].astype(jnp.int32)
    pad = _EPAD - _E
    dstp = jnp.concatenate(
        [dst, jnp.full((pad,), _N_NODES, jnp.int32)]).reshape(_NW, _CPT, _CH)
    pad2 = _EPAD2 - _E
    src2 = jnp.concatenate(
        [src, jnp.full((pad2,), _N_NODES, jnp.int32)]).reshape(-1, _CH)
    dst2 = jnp.concatenate(
        [dst, jnp.full((pad2,), _N_NODES, jnp.int32)]).reshape(-1, _CH)
    xp = jnp.pad(x, ((0, _NP - _N_NODES), (0, 0)))

    degp = _deg_kernel(dstp)
    xs = _prescale(degp, xp)
    p1 = _conv_kernel(xs, src2, dst2)
    h1s = _layer(_layer1_body, degp, p1, W1)
    p2 = _conv_kernel(h1s, src2, dst2)
    out = _layer(_layer2_body, degp, p2, W2)

    ue = _ugather_kernel(out, user_idx.astype(jnp.int32).reshape(_NW, _B // _NW))
    return ue, out[_N_USERS:_N_NODES]
